# Initial kernel scaffold; baseline (speedup 1.0000x reference)
import jax
import jax.numpy as jnp
from jax.experimental import pallas as pl


def kernel(x, edge_index, edge_attr, lin_w, mlp_w1, mlp_b1, mlp_w2, mlp_b2, conv_bias, prelu_a, gn_weight, gn_bias, gn_mean_scale):
    raise NotImplementedError("write your pallas kernel here")


# speedup vs baseline: 11.2985x; 11.2985x over previous
import functools

import jax
import jax.numpy as jnp
from jax import lax
from jax.experimental import pallas as pl
from jax.experimental.pallas import tpu as pltpu
from jax.experimental.pallas import tpu_sc as plsc

N = 10000
E = 320000
D = 128
D_EDGE = 16
D_HID = 32
EPS = 1e-5

NC = 2
NS = 16
NW = NC * NS
E_PER_TILE = E // NW
CHUNK = 80
ZROWS = 80
NZCH = N // ZROWS

def _sc_mesh():
    return plsc.VectorSubcoreMesh(
        core_axis_name="c", subcore_axis_name="s", num_cores=NC, num_subcores=NS
    )


def _edge_mlp_body(ea_ref, w1_ref, b1_ref, w2_ref, b2_ref, out_ref):
    ea = ea_ref[...]
    h = lax.dot_general(ea, w1_ref[...], (((1,), (1,)), ((), ())),
                        preferred_element_type=jnp.float32)
    h = jnp.maximum(h + b1_ref[...], 0.0)
    z = jnp.sum(h * w2_ref[...], axis=1) + b2_ref[0, 0]
    w = jax.nn.sigmoid(z)
    out_ref[...] = w.reshape(out_ref.shape)


def _edge_mlp(edge_attr, w1, b1, w2, b2):
    BE = 32000
    grid = E // BE
    out = pl.pallas_call(
        _edge_mlp_body,
        grid=(grid,),
        in_specs=[
            pl.BlockSpec((BE, D_EDGE), lambda i: (i, 0)),
            pl.BlockSpec((D_HID, D_EDGE), lambda i: (0, 0)),
            pl.BlockSpec((1, D_HID), lambda i: (0, 0)),
            pl.BlockSpec((1, D_HID), lambda i: (0, 0)),
            pl.BlockSpec((1, 1), lambda i: (0, 0)),
        ],
        out_specs=pl.BlockSpec((1, BE // 128, 128), lambda i: (i, 0, 0)),
        out_shape=jax.ShapeDtypeStruct((grid, BE // 128, 128), jnp.float32),
    )(edge_attr, w1, b1.reshape(1, D_HID), w2.reshape(1, D_HID), b2.reshape(1, 1))
    return out.reshape(E)


def _lin_body(x_ref, w_ref, out_ref):
    out_ref[...] = lax.dot_general(
        x_ref[...], w_ref[...], (((1,), (1,)), ((), ())),
        preferred_element_type=jnp.float32)


def _lin(x, lin_w):
    return pl.pallas_call(
        _lin_body,
        out_shape=jax.ShapeDtypeStruct((N, D), jnp.float32),
    )(x, lin_w)


def _deg_body(col_hbm, we_hbm, out_hbm, deg_v, col_v, we_v):
    cid = lax.axis_index("c")
    sid = lax.axis_index("s")
    wid = sid * NC + cid
    base = wid * E_PER_TILE

    zero = jnp.zeros((16,), jnp.float32)

    def zbody(i, _):
        deg_v[pl.ds(i * 16, 16)] = zero
        return 0

    lax.fori_loop(0, N // 16, zbody, 0)

    DC = 2000

    def body(ci, _):
        off = base + ci * DC
        pltpu.sync_copy(col_hbm.at[pl.ds(off, DC)], col_v.at[0])
        pltpu.sync_copy(we_hbm.at[pl.ds(off, DC)], we_v.at[0])

        def inner(k, _):
            idx = col_v[0, pl.ds(k * 16, 16)]
            w = we_v[0, pl.ds(k * 16, 16)]
            plsc.addupdate_scatter(deg_v, [idx], w)
            return 0

        lax.fori_loop(0, DC // 16, inner, 0)
        return 0

    lax.fori_loop(0, E_PER_TILE // DC, body, 0)
    pltpu.sync_copy(deg_v, out_hbm.at[wid])


def _deg_sc(col, w_e):
    DC = 2000
    f = pl.kernel(
        _deg_body,
        out_type=jax.ShapeDtypeStruct((NW, N), jnp.float32),
        mesh=_sc_mesh(),
        compiler_params=pltpu.CompilerParams(needs_layout_passes=False),
        scratch_types=[
            pltpu.VMEM((N,), jnp.float32),
            pltpu.VMEM((1, DC), jnp.int32),
            pltpu.VMEM((1, DC), jnp.float32),
        ],
    )
    return f(col, w_e)


def _combine_body(dp_ref, dis_ref, inv_ref):
    deg = 1.0 + jnp.sum(dp_ref[...], axis=0, keepdims=True)
    dis_ref[...] = lax.rsqrt(deg)
    inv_ref[...] = 1.0 / deg


def _combine(deg_partials):
    return pl.pallas_call(
        _combine_body,
        out_shape=(
            jax.ShapeDtypeStruct((1, N), jnp.float32),
            jax.ShapeDtypeStruct((1, N), jnp.float32),
        ),
    )(deg_partials)


def _agg_body(row_hbm, col_hbm, we_hbm, dis_hbm, h_hbm, out_hbm,
              dis_v, row_v, col_v, we_v, norm_v, rows_v, zbuf_v, acc_sh, sem):
    cid = lax.axis_index("c")
    sid = lax.axis_index("s")
    wid = sid * NC + cid
    base = wid * E_PER_TILE

    pltpu.sync_copy(dis_hbm, dis_v)

    zero = jnp.zeros((16,), jnp.float32)

    def zb(i, _):
        r = i // 8
        j = i % 8
        zbuf_v[r, pl.ds(j * 16, 16)] = zero
        return 0

    lax.fori_loop(0, ZROWS * 8, zb, 0)

    def zcopy(k, _):
        c = sid + k * NS

        @pl.when(c < NZCH)
        def _():
            pltpu.sync_copy(zbuf_v, acc_sh.at[pl.ds(c * ZROWS, ZROWS)])

        return 0

    lax.fori_loop(0, (NZCH + NS - 1) // NS, zcopy, 0)
    plsc.subcore_barrier()

    def body(ci, _):
        off = base + ci * CHUNK
        pltpu.sync_copy(row_hbm.at[pl.ds(off, CHUNK)], row_v.at[0])
        pltpu.sync_copy(col_hbm.at[pl.ds(off, CHUNK)], col_v.at[0])
        pltpu.sync_copy(we_hbm.at[pl.ds(off, CHUNK)], we_v.at[0])

        def nb(k, _):
            r = row_v[0, pl.ds(k * 16, 16)]
            c = col_v[0, pl.ds(k * 16, 16)]
            w = we_v[0, pl.ds(k * 16, 16)]
            nr = plsc.load_gather(dis_v, [r])
            nc_ = plsc.load_gather(dis_v, [c])
            norm_v[pl.ds(k * 16, 16)] = nr * w * nc_
            return 0

        lax.fori_loop(0, CHUNK // 16, nb, 0)

        pltpu.async_copy(h_hbm.at[row_v.at[0]], rows_v, sem).wait()

        def sb(e, _):
            s = plsc.load_gather(norm_v, [jnp.full((16,), e, dtype=jnp.int32)])
            for j in range(D // 16):
                rows_v[e, pl.ds(j * 16, 16)] = rows_v[e, pl.ds(j * 16, 16)] * s
            return 0

        lax.fori_loop(0, CHUNK, sb, 0)

        pltpu.sync_copy(rows_v, acc_sh.at[col_v.at[0]], add=True)
        return 0

    lax.fori_loop(0, E_PER_TILE // CHUNK, body, 0)
    plsc.subcore_barrier()

    def ocopy(k, _):
        c = sid + k * NS

        @pl.when(c < NZCH)
        def _():
            pltpu.sync_copy(acc_sh.at[pl.ds(c * ZROWS, ZROWS)],
                            out_hbm.at[cid, pl.ds(c * ZROWS, ZROWS)])

        return 0

    lax.fori_loop(0, (NZCH + NS - 1) // NS, ocopy, 0)


def _agg_sc(row, col, w_e, dis, h):
    f = pl.kernel(
        _agg_body,
        out_type=jax.ShapeDtypeStruct((NC, N, D), jnp.float32),
        mesh=_sc_mesh(),
        compiler_params=pltpu.CompilerParams(needs_layout_passes=False),
        scratch_types=[
            pltpu.VMEM((N,), jnp.float32),
            pltpu.VMEM((1, CHUNK), jnp.int32),
            pltpu.VMEM((1, CHUNK), jnp.int32),
            pltpu.VMEM((1, CHUNK), jnp.float32),
            pltpu.VMEM((CHUNK,), jnp.float32),
            pltpu.VMEM((CHUNK, D), jnp.float32),
            pltpu.VMEM((ZROWS, D), jnp.float32),
            pltpu.VMEM_SHARED((N, D), jnp.float32),
            pltpu.SemaphoreType.DMA,
        ],
    )
    return f(row, col, w_e, dis, h)


def _final_body(op_ref, h_ref, inv_ref, bias_ref, a_ref,
                gw_ref, gb_ref, gms_ref, out_ref):
    out = (op_ref[0] + op_ref[1]
           + inv_ref[...] * h_ref[...]
           + bias_ref[...])
    a = a_ref[0, 0]
    out = jnp.where(out >= 0.0, out, a * out)
    mean = jnp.mean(out, axis=0, keepdims=True)
    centered = out - mean * gms_ref[...]
    var = jnp.mean(centered * centered, axis=0, keepdims=True)
    std = jnp.sqrt(var + EPS)
    out_ref[...] = gw_ref[...] * centered / std + gb_ref[...]


def _final(out_p, h, inv_col, conv_bias, prelu_a, gn_weight, gn_bias, gn_mean_scale):
    return pl.pallas_call(
        _final_body,
        out_shape=jax.ShapeDtypeStruct((N, D), jnp.float32),
    )(out_p, h, inv_col, conv_bias.reshape(1, D), prelu_a.reshape(1, 1),
      gn_weight.reshape(1, D), gn_bias.reshape(1, D), gn_mean_scale.reshape(1, D))


def kernel(x, edge_index, edge_attr, lin_w, mlp_w1, mlp_b1, mlp_w2, mlp_b2,
           conv_bias, prelu_a, gn_weight, gn_bias, gn_mean_scale):
    row = edge_index[0]
    col = edge_index[1]
    w_e = _edge_mlp(edge_attr, mlp_w1, mlp_b1, mlp_w2, mlp_b2)
    h = _lin(x, lin_w)
    deg_partials = _deg_sc(col, w_e)
    dis2d, inv2d = _combine(deg_partials)
    dis = dis2d.reshape(N)
    inv_col = inv2d.reshape(N, 1)
    out_p = _agg_sc(row, col, w_e, dis, h)
    return _final(out_p, h, inv_col, conv_bias, prelu_a,
                  gn_weight, gn_bias, gn_mean_scale)
